# Initial kernel scaffold; baseline (speedup 1.0000x reference)
import jax
import jax.numpy as jnp
from jax.experimental import pallas as pl


def kernel(adj_values, uEmbeds, iEmbeds, edge_index):
    raise NotImplementedError("write your pallas kernel here")


# speedup vs baseline: 3.6451x; 3.6451x over previous
import functools

import jax
import jax.numpy as jnp
from jax import lax
from jax.experimental import pallas as pl
from jax.experimental.pallas import tpu as pltpu
from jax.experimental.pallas import tpu_sc as plsc

USER = 5000
ITEM = 5000
N = USER + ITEM
E = 320000
D = 128
NUM_LAYERS = 3

NC = 2
NS = 16
NW = NC * NS
EPW = E // NW
CH = 80
NCHUNK = EPW // CH
RPT = 624
TAIL = N - NS * RPT
ZR = 104

_mesh = plsc.VectorSubcoreMesh(core_axis_name="c", subcore_axis_name="s")


def _spmm_body(cur_hbm, col_hbm, row_hbm, val_hbm, out_hbm,
               idx_v, ridx_v, val_v, rows_v, zero_v, acc_sh):
    c = lax.axis_index("c")
    s = lax.axis_index("s")
    wid = s * NC + c

    z16 = jnp.zeros((16,), jnp.float32)

    def zero_row(r, _):
        for j in range(D // 16):
            zero_v[r, pl.ds(j * 16, 16)] = z16
        return 0

    lax.fori_loop(0, ZR, zero_row, 0)
    for k in range(RPT // ZR):
        pltpu.sync_copy(zero_v, acc_sh.at[pl.ds(s * RPT + k * ZR, ZR)])

    @pl.when(s == NS - 1)
    def _zero_tail():
        pltpu.sync_copy(zero_v.at[pl.ds(0, TAIL)],
                        acc_sh.at[pl.ds(NS * RPT, TAIL)])

    plsc.subcore_barrier()

    def chunk_body(i, _):
        base = wid * EPW + i * CH
        pltpu.sync_copy(col_hbm.at[pl.ds(base, CH)], idx_v)
        pltpu.sync_copy(row_hbm.at[pl.ds(base, CH)], ridx_v)
        pltpu.sync_copy(val_hbm.at[pl.ds(base, CH)], val_v)
        pltpu.sync_copy(cur_hbm.at[idx_v], rows_v)

        def mul_body(g, _):
            vv = val_v[pl.ds(g * 16, 16)]
            for i in range(16):
                splat = jnp.full((16,), vv[i], jnp.float32)
                e = g * 16 + i
                for j in range(D // 16):
                    sl = pl.ds(j * 16, 16)
                    rows_v[e, sl] = rows_v[e, sl] * splat
            return 0

        lax.fori_loop(0, CH // 16, mul_body, 0)
        pltpu.sync_copy(rows_v, acc_sh.at[ridx_v], add=True)
        return 0

    lax.fori_loop(0, NCHUNK, chunk_body, 0)
    plsc.subcore_barrier()

    pltpu.sync_copy(acc_sh.at[pl.ds(s * RPT, RPT)],
                    out_hbm.at[c, pl.ds(s * RPT, RPT)])

    @pl.when(s == NS - 1)
    def _write_tail():
        pltpu.sync_copy(acc_sh.at[pl.ds(NS * RPT, TAIL)],
                        out_hbm.at[c, pl.ds(NS * RPT, TAIL)])


_spmm = pl.kernel(
    _spmm_body,
    out_type=jax.ShapeDtypeStruct((NC, N, D), jnp.float32),
    mesh=_mesh,
    scratch_types=[
        pltpu.VMEM((CH,), jnp.int32),
        pltpu.VMEM((CH,), jnp.int32),
        pltpu.VMEM((CH,), jnp.float32),
        pltpu.VMEM((CH, D), jnp.float32),
        pltpu.VMEM((ZR, D), jnp.float32),
        pltpu.VMEM_SHARED((N, D), jnp.float32),
    ],
)


def _combine_body(p_ref, t_ref, cur_ref, tot_ref):
    layer = p_ref[0] + p_ref[1]
    cur_ref[...] = layer
    tot_ref[...] = t_ref[...] + layer


_BR = 2000


def _combine(part, tot):
    grid = (N // _BR,)
    return pl.pallas_call(
        _combine_body,
        grid=grid,
        in_specs=[
            pl.BlockSpec((NC, _BR, D), lambda i: (0, i, 0)),
            pl.BlockSpec((_BR, D), lambda i: (i, 0)),
        ],
        out_specs=[
            pl.BlockSpec((_BR, D), lambda i: (i, 0)),
            pl.BlockSpec((_BR, D), lambda i: (i, 0)),
        ],
        out_shape=[
            jax.ShapeDtypeStruct((N, D), jnp.float32),
            jax.ShapeDtypeStruct((N, D), jnp.float32),
        ],
    )(part, tot)


def kernel(adj_values, uEmbeds, iEmbeds, edge_index):
    embeds = jnp.concatenate([uEmbeds, iEmbeds], axis=0)
    row = edge_index[0].astype(jnp.int32)
    col = edge_index[1].astype(jnp.int32)
    vals = adj_values.astype(jnp.float32)

    cur = embeds
    tot = embeds
    for _ in range(NUM_LAYERS):
        part = _spmm(cur, col, row, vals)
        cur, tot = _combine(part, tot)
    return tot[:USER], tot[USER:]


# speedup vs baseline: 8.4105x; 2.3074x over previous
import functools

import jax
import jax.numpy as jnp
from jax import lax
from jax.experimental import pallas as pl
from jax.experimental.pallas import tpu as pltpu
from jax.experimental.pallas import tpu_sc as plsc

USER = 5000
ITEM = 5000
N = USER + ITEM
E = 320000
D = 128
NUM_LAYERS = 3

NC = 2
NS = 16
NW = NC * NS
EPW = E // NW
CH = 80
NCHUNK = EPW // CH
RPT = 624
TAIL = N - NS * RPT
ZCH = 80

_mesh = plsc.VectorSubcoreMesh(core_axis_name="c", subcore_axis_name="s")


def _spmm_body(cur_hbm, pk_hbm, val_hbm, out_hbm,
               rows0, rows1, pk0, pk1, ridx0, ridx1, valv, acc_sh,
               gsem0, gsem1, ssem0, ssem1, pksem0, pksem1):
    c = lax.axis_index("c")
    s = lax.axis_index("s")
    wid = s * NC + c
    rows = (rows0, rows1)
    pkv = (pk0, pk1)
    ridx = (ridx0, ridx1)
    gsem = (gsem0, gsem1)
    ssem = (ssem0, ssem1)
    pksem = (pksem0, pksem1)

    z16 = jnp.zeros((16,), jnp.float32)

    def zero_row(r, _):
        for j in range(D // 16):
            rows0[r, pl.ds(j * 16, 16)] = z16
        return 0

    lax.fori_loop(0, ZCH, zero_row, 0)
    for k in range(RPT // ZCH):
        pltpu.sync_copy(rows0, acc_sh.at[pl.ds(s * RPT + k * ZCH, ZCH)])
    rem = RPT - (RPT // ZCH) * ZCH
    if rem:
        pltpu.sync_copy(rows0.at[pl.ds(0, rem), :],
                        acc_sh.at[pl.ds(s * RPT + (RPT // ZCH) * ZCH, rem)])

    @pl.when(s == NS - 1)
    def _zero_tail():
        pltpu.sync_copy(rows0.at[pl.ds(0, TAIL), :],
                        acc_sh.at[pl.ds(NS * RPT, TAIL)])

    plsc.subcore_barrier()

    def start_pk(i, b):
        pltpu.async_copy(pk_hbm.at[wid, i], pkv[b], pksem[b])

    def wait_pk(b):
        pltpu.make_async_copy(pk_hbm.at[wid, 0], pkv[b], pksem[b]).wait()

    def start_gather(b):
        pltpu.async_copy(cur_hbm.at[pkv[b].at[0]], rows[b], gsem[b])

    def wait_gather(b):
        pltpu.make_async_copy(cur_hbm.at[pkv[b].at[0]], rows[b],
                              gsem[b]).wait()

    def start_scatter(b):
        pltpu.async_copy(rows[b], acc_sh.at[ridx[b]], ssem[b], add=True)

    def wait_scatter(b):
        pltpu.make_async_copy(rows[b], acc_sh.at[ridx[b]], ssem[b]).wait()

    def copy_ridx(b):
        for g in range(CH // 16):
            sl = pl.ds(g * 16, 16)
            ridx[b][sl] = pkv[b][1, sl]

    def mul(b, i):
        def mul_body(g, _):
            vv = valv[i, pl.ds(g * 16, 16)]
            for k in range(16):
                splat = jnp.full((16,), vv[k], jnp.float32)
                e = g * 16 + k
                for j in range(D // 16):
                    sl = pl.ds(j * 16, 16)
                    rows[b][e, sl] = rows[b][e, sl] * splat
            return 0

        lax.fori_loop(0, CH // 16, mul_body, 0)

    pltpu.sync_copy(val_hbm.at[wid], valv)
    pltpu.sync_copy(pk_hbm.at[wid, 0], pkv[0])
    start_pk(1, 1)
    start_gather(0)

    def outer_body(t, _):
        for b in (0, 1):
            i = 2 * t + b
            wait_gather(b)
            copy_ridx(b)
            wait_pk(1 - b)
            if b == 0:
                @pl.when(t >= 1)
                def _w():
                    wait_scatter(1)
            else:
                wait_scatter(0)
            start_gather(1 - b)
            mul(b, i)
            start_scatter(b)
            if b == 0:
                start_pk(i + 2, b)
            else:
                @pl.when(t < (NCHUNK - 3) // 2)
                def _p():
                    start_pk(i + 2, b)
        return 0

    lax.fori_loop(0, (NCHUNK - 1) // 2, outer_body, 0)

    wait_gather(0)
    copy_ridx(0)
    wait_scatter(1)
    mul(0, NCHUNK - 1)
    start_scatter(0)
    wait_scatter(0)
    plsc.subcore_barrier()

    pltpu.sync_copy(acc_sh.at[pl.ds(s * RPT, RPT)],
                    out_hbm.at[c, pl.ds(s * RPT, RPT)])

    @pl.when(s == NS - 1)
    def _write_tail():
        pltpu.sync_copy(acc_sh.at[pl.ds(NS * RPT, TAIL)],
                        out_hbm.at[c, pl.ds(NS * RPT, TAIL)])


_spmm = pl.kernel(
    _spmm_body,
    out_type=jax.ShapeDtypeStruct((NC, N, D), jnp.float32),
    mesh=_mesh,
    scratch_types=[
        pltpu.VMEM((CH, D), jnp.float32),
        pltpu.VMEM((CH, D), jnp.float32),
        pltpu.VMEM((2, CH), jnp.int32),
        pltpu.VMEM((2, CH), jnp.int32),
        pltpu.VMEM((CH,), jnp.int32),
        pltpu.VMEM((CH,), jnp.int32),
        pltpu.VMEM((NCHUNK, CH), jnp.float32),
        pltpu.VMEM_SHARED((N, D), jnp.float32),
        pltpu.SemaphoreType.DMA,
        pltpu.SemaphoreType.DMA,
        pltpu.SemaphoreType.DMA,
        pltpu.SemaphoreType.DMA,
        pltpu.SemaphoreType.DMA,
        pltpu.SemaphoreType.DMA,
    ],
)


def _combine_body(p_ref, t_ref, cur_ref, tot_ref):
    layer = p_ref[0] + p_ref[1]
    cur_ref[...] = layer
    tot_ref[...] = t_ref[...] + layer


_BR = 2000


def _combine(part, tot):
    grid = (N // _BR,)
    return pl.pallas_call(
        _combine_body,
        grid=grid,
        in_specs=[
            pl.BlockSpec((NC, _BR, D), lambda i: (0, i, 0)),
            pl.BlockSpec((_BR, D), lambda i: (i, 0)),
        ],
        out_specs=[
            pl.BlockSpec((_BR, D), lambda i: (i, 0)),
            pl.BlockSpec((_BR, D), lambda i: (i, 0)),
        ],
        out_shape=[
            jax.ShapeDtypeStruct((N, D), jnp.float32),
            jax.ShapeDtypeStruct((N, D), jnp.float32),
        ],
    )(part, tot)


def kernel(adj_values, uEmbeds, iEmbeds, edge_index):
    embeds = jnp.concatenate([uEmbeds, iEmbeds], axis=0)
    row = edge_index[0].astype(jnp.int32).reshape(NW, NCHUNK, 1, CH)
    col = edge_index[1].astype(jnp.int32).reshape(NW, NCHUNK, 1, CH)
    pk = jnp.concatenate([col, row], axis=2)
    vals = adj_values.astype(jnp.float32).reshape(NW, NCHUNK, CH)

    cur = embeds
    tot = embeds
    for _ in range(NUM_LAYERS):
        part = _spmm(cur, pk, vals)
        cur, tot = _combine(part, tot)
    return tot[:USER], tot[USER:]
